# Initial kernel scaffold; baseline (speedup 1.0000x reference)
import functools

import jax
import jax.numpy as jnp
from jax import lax
from jax.experimental import pallas as pl
from jax.experimental.pallas import tpu as pltpu
from jax.experimental.pallas import tpu_sc as plsc

B = 4096
L = 200
E = 32
NC = 2
NS = 16
NW = NC * NS
BPW = B // NW
NBUF = 2
HALF = L // 2


def _sc_body(x_hbm, w_hbm, out_hbm, idx_v, rows_v, outt_v, gsem, osem):
  wid = lax.axis_index("s") * NC + lax.axis_index("c")
  base = wid * BPW

  iota = lax.iota(jnp.int32, 16)
  c0 = iota * L
  c1 = c0 + 16 * L

  def start_gather(i, k):
    b = base + i
    pltpu.sync_copy(x_hbm.at[b], idx_v.at[k])
    for j in range(2):
      pltpu.async_copy(
          w_hbm.at[idx_v.at[k].at[j]],
          rows_v.at[k].at[pl.ds(j * HALF, HALF)],
          gsem.at[k],
      )

  def wait_gather(k):
    for j in range(2):
      pltpu.make_async_copy(
          w_hbm.at[idx_v.at[k].at[j]],
          rows_v.at[k].at[pl.ds(j * HALF, HALF)],
          gsem.at[k],
      ).wait()

  def transpose(k):
    rows = rows_v.at[k]
    outt = outt_v.at[k]

    def step(i, _):
      for u in range(4):
        l = i * 4 + u
        v0 = rows[l, 0:16]
        v1 = rows[l, 16:32]
        plsc.store_scatter(outt, [c0 + l], v0)
        plsc.store_scatter(outt, [c1 + l], v1)
      return 0

    lax.fori_loop(0, L // 4, step, 0)

  for k in range(NBUF):
    start_gather(k, k)

  def outer(g, _):
    for k in range(NBUF):
      i = g * NBUF + k
      b = base + i
      wait_gather(k)

      @pl.when(g > 0)
      def _():
        pltpu.make_async_copy(outt_v.at[k], out_hbm.at[b], osem.at[k]).wait()

      transpose(k)
      pltpu.async_copy(outt_v.at[k], out_hbm.at[b], osem.at[k])

      @pl.when(i + NBUF < BPW)
      def _():
        start_gather(i + NBUF, k)

    return 0

  lax.fori_loop(0, BPW // NBUF, outer, 0)

  for k in range(NBUF):
    pltpu.make_async_copy(outt_v.at[k], out_hbm.at[base], osem.at[k]).wait()


@jax.jit
def kernel(x, W):
  x = x.astype(jnp.int32).reshape(B, 2, HALF)
  run = pl.kernel(
      _sc_body,
      out_type=jax.ShapeDtypeStruct((B, E * L), jnp.float32),
      mesh=plsc.VectorSubcoreMesh(core_axis_name="c", subcore_axis_name="s"),
      scratch_types=[
          pltpu.VMEM((NBUF, 2, HALF), jnp.int32),
          pltpu.VMEM((NBUF, L, E), jnp.float32),
          pltpu.VMEM((NBUF, E * L), jnp.float32),
          pltpu.SemaphoreType.DMA((NBUF,)),
          pltpu.SemaphoreType.DMA((NBUF,)),
      ],
  )
  out = run(x, W)
  return out.reshape(B, E, L)


# speedup vs baseline: 5.0304x; 5.0304x over previous
import jax
import jax.numpy as jnp
from jax import lax
from jax.experimental import pallas as pl
from jax.experimental.pallas import tpu as pltpu
from jax.experimental.pallas import tpu_sc as plsc

B = 4096
L = 200
E = 32
NC = 2
NS = 16
NW = NC * NS
BPW = B // NW
NBUF = 2
HALF = L // 2


def _sc_body(x_hbm, w_hbm, out_hbm, idx_v, rows_v, outts, gsem, osem):
  wid = lax.axis_index("s") * NC + lax.axis_index("c")
  base = wid * BPW

  iota = lax.iota(jnp.int32, 16)
  e_lo = iota
  e_hi = iota + 16

  pltpu.sync_copy(x_hbm.at[pl.ds(base, BPW)], idx_v)

  def start_gather(i, k):
    for j in range(2):
      pltpu.async_copy(
          w_hbm.at[idx_v.at[i].at[j]],
          rows_v.at[k].at[pl.ds(j * HALF, HALF)],
          gsem.at[k],
      )

  def wait_gather(i, k):
    for j in range(2):
      pltpu.make_async_copy(
          w_hbm.at[idx_v.at[i].at[j]],
          rows_v.at[k].at[pl.ds(j * HALF, HALF)],
          gsem.at[k],
      ).wait()

  def transpose(k):
    rows = rows_v.at[k]
    outt = outts[k]

    def step(l, _):
      lv = jnp.full((16,), l, jnp.int32)
      plsc.store_scatter(outt, [e_lo, lv], rows[l, 0:16])
      plsc.store_scatter(outt, [e_hi, lv], rows[l, 16:32])
      return 0

    lax.fori_loop(0, L, step, 0)

  for k in range(NBUF):
    start_gather(k, k)

  def outer(g, _):
    for k in range(NBUF):
      i = g * NBUF + k
      b = base + i
      wait_gather(i, k)

      @pl.when(g > 0)
      def _():
        pltpu.make_async_copy(outts[k], out_hbm.at[b], osem.at[k]).wait()

      transpose(k)
      pltpu.async_copy(outts[k], out_hbm.at[b], osem.at[k])

      @pl.when(i + NBUF < BPW)
      def _():
        start_gather(i + NBUF, k)

    return 0

  lax.fori_loop(0, BPW // NBUF, outer, 0)

  for k in range(NBUF):
    pltpu.make_async_copy(outts[k], out_hbm.at[base], osem.at[k]).wait()


def _sc_entry(x_hbm, w_hbm, out_hbm, idx_v, rows_v, outt_0, outt_1, gsem,
              osem):
  _sc_body(x_hbm, w_hbm, out_hbm, idx_v, rows_v, (outt_0, outt_1), gsem, osem)


@jax.jit
def kernel(x, W):
  x = x.astype(jnp.int32).reshape(B, 2, HALF)
  run = pl.kernel(
      _sc_entry,
      out_type=jax.ShapeDtypeStruct((B, E, L), jnp.float32),
      mesh=plsc.VectorSubcoreMesh(core_axis_name="c", subcore_axis_name="s"),
      compiler_params=pltpu.CompilerParams(
          use_tc_tiling_on_sc=False, needs_layout_passes=False
      ),
      scratch_types=[
          pltpu.VMEM((BPW, 2, HALF), jnp.int32),
          pltpu.VMEM((NBUF, L, E), jnp.float32),
          pltpu.VMEM((E, L), jnp.float32),
          pltpu.VMEM((E, L), jnp.float32),
          pltpu.SemaphoreType.DMA((NBUF,)),
          pltpu.SemaphoreType.DMA((NBUF,)),
      ],
  )
  return run(x, W)
